# baseline (speedup 1.0000x reference)
import functools
import jax
import jax.numpy as jnp
from jax import lax
from jax.experimental import pallas as pl
from jax.experimental.pallas import tpu as pltpu

B = 32
Q = 64
F = 64
H = 2
NC = 1024
NCN = B * NC
ECG = 16384
ETG = 512


def _lrelu(x, a):
    return jnp.maximum(x, a * x)


def _proj_body(num_ref, den_ref, bias_ref, wl_ref, bl_ref, wr_ref, br_ref,
               xl_ref, xr_ref, *, h_prev, act):
    x = num_ref[...]
    if act:
        den = den_ref[...]
        c = x.shape[1] // h_prev
        den_rep = jnp.concatenate(
            [jnp.broadcast_to(den[:, i:i + 1], (x.shape[0], c))
             for i in range(h_prev)], axis=1)
        x = _lrelu(x / den_rep + bias_ref[...], 0.01)
    xl_ref[...] = jnp.dot(x, wl_ref[...],
                          preferred_element_type=jnp.float32) + bl_ref[...]
    xr_ref[...] = jnp.dot(x, wr_ref[...],
                          preferred_element_type=jnp.float32) + br_ref[...]


def _proj(x, den, bias_prev, Wl, bl, Wr, br, h_prev, act):
    N, din = x.shape
    dout = Wl.shape[1]
    BLK = 512
    bias2 = bias_prev.reshape(1, din)
    bl2 = bl.reshape(1, dout)
    br2 = br.reshape(1, dout)
    grid = (N // BLK,)
    body = functools.partial(_proj_body, h_prev=h_prev, act=act)
    return pl.pallas_call(
        body,
        grid=grid,
        in_specs=[
            pl.BlockSpec((BLK, din), lambda i: (i, 0)),
            pl.BlockSpec((BLK, h_prev), lambda i: (i, 0)),
            pl.BlockSpec((1, din), lambda i: (0, 0)),
            pl.BlockSpec((din, dout), lambda i: (0, 0)),
            pl.BlockSpec((1, dout), lambda i: (0, 0)),
            pl.BlockSpec((din, dout), lambda i: (0, 0)),
            pl.BlockSpec((1, dout), lambda i: (0, 0)),
        ],
        out_specs=[
            pl.BlockSpec((BLK, dout), lambda i: (i, 0)),
            pl.BlockSpec((BLK, dout), lambda i: (i, 0)),
        ],
        out_shape=[
            jax.ShapeDtypeStruct((N, dout), jnp.float32),
            jax.ShapeDtypeStruct((N, dout), jnp.float32),
        ],
    )(x, den, bias2, Wl, bl2, Wr, br2)


def _ln_body(num_ref, den_ref, bias_ref, x_ref, w_ref, b_ref, o_ref):
    x = x_ref[...]
    g = _lrelu(num_ref[...] / den_ref[...] + bias_ref[...], 0.01)
    t = g + x
    mu = jnp.mean(t)
    var = jnp.mean((t - mu) * (t - mu))
    o_ref[...] = (t - mu) / jnp.sqrt(var + 1e-5) * w_ref[...] + b_ref[...] + x


def _graph_ln_res(num, den, bias, x, w, b, npg):
    N, fd = x.shape
    grid = (N // npg,)
    return pl.pallas_call(
        _ln_body,
        grid=grid,
        in_specs=[
            pl.BlockSpec((npg, fd), lambda i: (i, 0)),
            pl.BlockSpec((npg, 1), lambda i: (i, 0)),
            pl.BlockSpec((1, fd), lambda i: (0, 0)),
            pl.BlockSpec((npg, fd), lambda i: (i, 0)),
            pl.BlockSpec((1, fd), lambda i: (0, 0)),
            pl.BlockSpec((1, fd), lambda i: (0, 0)),
        ],
        out_specs=pl.BlockSpec((npg, fd), lambda i: (i, 0)),
        out_shape=jax.ShapeDtypeStruct((N, fd), jnp.float32),
    )(num, den, bias.reshape(1, fd), x, w.reshape(1, fd), b.reshape(1, fd))


def _mlp_body(zp_ref, w1_ref, b1_ref, w2_ref, b2_ref, lnw_ref, lnb_ref,
              w3_ref, b3_ref, w4_ref, b4_ref, o_ref):
    x = zp_ref[...]
    h1 = _lrelu(jnp.dot(x, w1_ref[...],
                        preferred_element_type=jnp.float32) + b1_ref[...], 0.01)
    h2 = _lrelu(jnp.dot(h1, w2_ref[...],
                        preferred_element_type=jnp.float32) + b2_ref[...], 0.01)
    mu = jnp.mean(h2, axis=1, keepdims=True)
    va = jnp.mean((h2 - mu) * (h2 - mu), axis=1, keepdims=True)
    h2 = (h2 - mu) / jnp.sqrt(va + 1e-5) * lnw_ref[...] + lnb_ref[...]
    h3 = _lrelu(jnp.dot(h2, w3_ref[...],
                        preferred_element_type=jnp.float32) + b3_ref[...], 0.01)
    o_ref[...] = jnp.dot(h3, w4_ref[...],
                         preferred_element_type=jnp.float32) + b4_ref[...]


def _mlp(zp, mp):
    R = zp.shape[0]
    BLK = 2048
    d1 = 6 * F
    d2 = 2 * F
    d3 = F // 2
    W4p = jnp.concatenate([mp["W4"], jnp.zeros((d3, 7), jnp.float32)], axis=1)
    b4p = jnp.concatenate([mp["b4"], jnp.zeros((7,), jnp.float32)])
    return pl.pallas_call(
        _mlp_body,
        grid=(R // BLK,),
        in_specs=[
            pl.BlockSpec((BLK, d1), lambda i: (i, 0)),
            pl.BlockSpec((d1, d1), lambda i: (0, 0)),
            pl.BlockSpec((1, d1), lambda i: (0, 0)),
            pl.BlockSpec((d1, d2), lambda i: (0, 0)),
            pl.BlockSpec((1, d2), lambda i: (0, 0)),
            pl.BlockSpec((1, d2), lambda i: (0, 0)),
            pl.BlockSpec((1, d2), lambda i: (0, 0)),
            pl.BlockSpec((d2, d3), lambda i: (0, 0)),
            pl.BlockSpec((1, d3), lambda i: (0, 0)),
            pl.BlockSpec((d3, 8), lambda i: (0, 0)),
            pl.BlockSpec((1, 8), lambda i: (0, 0)),
        ],
        out_specs=pl.BlockSpec((BLK, 8), lambda i: (i, 0)),
        out_shape=jax.ShapeDtypeStruct((R, 8), jnp.float32),
    )(zp, mp["W1"], mp["b1"].reshape(1, d1), mp["W2"], mp["b2"].reshape(1, d2),
      mp["lnw"].reshape(1, d2), mp["lnb"].reshape(1, d2), mp["W3"],
      mp["b3"].reshape(1, d3), W4p, b4p.reshape(1, 8))


def _sc_edge(xl, xr, srcl, dstl, attf, h, npg, epg, ngraphs):
    from jax.experimental.pallas import tpu_sc as plsc

    N, D = xl.shape
    C = D // h
    NCH = D // 16
    MC = -(-NCH // 16)
    Eg = epg
    ES = -(-Eg // 16 // 16) * 16
    EgP = 16 * ES
    T = max(1, Eg // 1088)
    EB = Eg // T
    W = EgP + 256
    G = ngraphs // 2
    f32, i32 = jnp.float32, jnp.int32

    mesh = plsc.VectorSubcoreMesh(core_axis_name="c", subcore_axis_name="s")
    scratch = ([pltpu.VMEM((npg, 16), f32)] * MC
               + [pltpu.VMEM((npg, 16), f32)] * MC
               + [pltpu.VMEM((EB + 16,), i32),
                  pltpu.VMEM((EB + 16,), i32),
                  pltpu.VMEM((EB,), f32),
                  pltpu.VMEM((h, W), f32),
                  pltpu.VMEM((ES,), f32),
                  pltpu.VMEM((ES,), f32)]
               + [pltpu.VMEM((npg, 16), f32)] * MC
               + [pltpu.VMEM((npg,), f32),
                  pltpu.SMEM((D,), f32),
                  pltpu.VMEM_SHARED((16, h, EgP), f32),
                  pltpu.VMEM_SHARED((h, W), f32),
                  pltpu.VMEM_SHARED((D,), f32)])

    def body(xl_hbm, xr_hbm, src_hbm, dst_hbm, att_hbm, num_hbm, den_hbm,
             *scr):
        xlb = scr[0:MC]
        xrb = scr[MC:2 * MC]
        srcb, dstb, plg, lgb, tmp, csum = scr[2 * MC:2 * MC + 6]
        accb = scr[2 * MC + 6:3 * MC + 6]
        denb, att_s, parts, wbuf, att_sp = scr[3 * MC + 6:]

        cc_ax = lax.axis_index("c")
        s_ax = lax.axis_index("s")
        iota = lax.iota(i32, 16)
        z16 = jnp.zeros((16,), f32)

        @pl.when(s_ax == 0)
        def _():
            pltpu.sync_copy(att_hbm, att_sp)
        plsc.subcore_barrier()
        pltpu.sync_copy(att_sp, att_s)

        def per_graph(gi, _):
            b = cc_ax * G + gi
            for k in range(MC):
                ch = s_ax + 16 * k

                @pl.when(ch < NCH)
                def _(k=k, ch=ch):
                    pltpu.sync_copy(
                        xl_hbm.at[pl.ds(b * npg, npg), pl.ds(ch * 16, 16)],
                        xlb[k])
                    pltpu.sync_copy(
                        xr_hbm.at[pl.ds(b * npg, npg), pl.ds(ch * 16, 16)],
                        xrb[k])

                def zacc(i, _, k=k):
                    accb[k][i, :] = z16
                    return 0
                lax.fori_loop(0, npg, zacc, 0)

            def zden(i, _):
                denb[pl.ds(i * 16, 16)] = z16
                return 0
            lax.fori_loop(0, npg // 16, zden, 0)

            def passA(t, _):
                e0 = b * Eg + t * EB
                pltpu.sync_copy(src_hbm.at[pl.ds(e0, EB)],
                                srcb.at[pl.ds(0, EB)])
                pltpu.sync_copy(dst_hbm.at[pl.ds(e0, EB)],
                                dstb.at[pl.ds(0, EB)])
                for k in range(MC):
                    ch = s_ax + 16 * k

                    @pl.when(ch < NCH)
                    def _(k=k, ch=ch):
                        def grp(i, _):
                            sv = srcb[pl.ds(i * 16, 16)]
                            dv = dstb[pl.ds(i * 16, 16)]
                            acc = z16
                            for ccol in range(16):
                                cidx = jnp.zeros((16,), i32) + ccol
                                gl = plsc.load_gather(xlb[k], [sv, cidx])
                                gr = plsc.load_gather(xrb[k], [dv, cidx])
                                t0 = gl + gr
                                t0 = jnp.maximum(t0, 0.2 * t0)
                                acc = acc + t0 * att_s[ch * 16 + ccol]
                            plg[pl.ds(i * 16, 16)] = acc
                            return 0
                        lax.fori_loop(0, EB // 16, grp, 0)
                        hd = (ch * 16) // C
                        pltpu.sync_copy(plg,
                                        parts.at[s_ax, hd,
                                                 pl.ds(t * EB, EB)])

                return 0
            lax.fori_loop(0, T, passA, 0)
            plsc.subcore_barrier()

            owners = [[j for j in range(16)
                       if any(j + 16 * k < NCH
                              and ((j + 16 * k) * 16) // C == hd
                              for k in range(MC))]
                      for hd in range(h)]
            es0 = s_ax * ES
            nv = jnp.clip((Eg - es0) // 16, 0, ES // 16)
            for hh in range(h):
                def zc(i, _):
                    csum[pl.ds(i * 16, 16)] = z16
                    return 0
                lax.fori_loop(0, ES // 16, zc, 0)
                for j in owners[hh]:
                    pltpu.sync_copy(parts.at[j, hh, pl.ds(es0, ES)], tmp)

                    def add16(i, _):
                        csum[pl.ds(i * 16, 16)] = (csum[pl.ds(i * 16, 16)]
                                                   + tmp[pl.ds(i * 16, 16)])
                        return 0
                    lax.fori_loop(0, ES // 16, add16, 0)
                pltpu.sync_copy(csum, wbuf.at[hh, pl.ds(es0, ES)])

                def mx(i, m):
                    return jnp.maximum(m, csum[pl.ds(i * 16, 16)])
                mv = lax.fori_loop(0, nv, mx, jnp.full((16,), -3e38, f32))
                plg[pl.ds(0, 16)] = jnp.zeros((16,), f32) + jnp.max(mv)
                pltpu.sync_copy(plg.at[pl.ds(0, 16)],
                                wbuf.at[hh, pl.ds(EgP + s_ax * 16, 16)])
            plsc.subcore_barrier()

            pltpu.sync_copy(wbuf, lgb)
            for hh in range(h):
                def mx2(i, m, hh=hh):
                    return jnp.maximum(m, lgb[hh, pl.ds(EgP + i * 16, 16)])
                mv = lax.fori_loop(0, 16, mx2, jnp.full((16,), -3e38, f32))
                mh = jnp.max(mv)

                def ex(i, _, hh=hh, mh=mh):
                    v = lgb[hh, pl.ds(i * 16, 16)]
                    lgb[hh, pl.ds(i * 16, 16)] = jnp.exp(v - mh)
                    return 0
                lax.fori_loop(0, Eg // 16, ex, 0)
            plsc.subcore_barrier()

            def passB(t, _):
                e0 = b * Eg + t * EB
                pltpu.sync_copy(src_hbm.at[pl.ds(e0, EB)],
                                srcb.at[pl.ds(0, EB)])
                pltpu.sync_copy(dst_hbm.at[pl.ds(e0, EB)],
                                dstb.at[pl.ds(0, EB)])

                def grp(i, _):
                    sv = srcb[pl.ds(i * 16, 16)]
                    dv = dstb[pl.ds(i * 16, 16)]
                    for k in range(MC):
                        ch = s_ax + 16 * k

                        @pl.when(ch < NCH)
                        def _(k=k, ch=ch):
                            hd = (ch * 16) // C
                            wv = lgb[hd, pl.ds(t * EB + i * 16, 16)]
                            for ccol in range(16):
                                cidx = jnp.zeros((16,), i32) + ccol
                                gl = plsc.load_gather(xlb[k], [sv, cidx])
                                plsc.addupdate_scatter(accb[k], [dv, cidx],
                                                       gl * wv)

                            @pl.when(ch * 16 % C == 0)
                            def _():
                                plsc.addupdate_scatter(denb, [dv], wv)
                    return 0
                lax.fori_loop(0, EB // 16, grp, 0)
                return 0
            lax.fori_loop(0, T, passB, 0)

            for k in range(MC):
                ch = s_ax + 16 * k

                @pl.when(ch < NCH)
                def _(k=k, ch=ch):
                    pltpu.sync_copy(
                        accb[k],
                        num_hbm.at[pl.ds(b * npg, npg), pl.ds(ch * 16, 16)])

                    @pl.when(ch * 16 % C == 0)
                    def _():
                        hd = (ch * 16) // C
                        pltpu.sync_copy(denb,
                                        den_hbm.at[hd, pl.ds(b * npg, npg)])
            plsc.subcore_barrier()
            return 0

        lax.fori_loop(0, G, per_graph, 0)

    run = pl.kernel(
        body,
        out_type=[jax.ShapeDtypeStruct((N, D), f32),
                  jax.ShapeDtypeStruct((h, N), f32)],
        mesh=mesh,
        scratch_types=scratch,
        compiler_params=pltpu.CompilerParams(use_tc_tiling_on_sc=False,
                                             needs_layout_passes=False),
    )
    num, den = run(xl, xr, srcl, dstl, attf)
    return num, den.T


def _edge_stage(xl, xr, srcl, dstl, att, h, npg, epg):
    ngraphs = xl.shape[0] // npg
    return _sc_edge(xl, xr, srcl, dstl, att.reshape(-1), h, npg, epg,
                    ngraphs)


def _stack(x0, srcl, dstl, sp, fd, h, npg, epg):
    num, den, bias = x0, jnp.ones((x0.shape[0], 1), jnp.float32), None
    h_prev, act = 1, False
    for li, p in enumerate(sp["convs"]):
        hh = h if li < 4 else 1
        bias_in = bias if bias is not None else jnp.zeros((num.shape[1],),
                                                          jnp.float32)
        xl, xr = _proj(num, den, bias_in, p["Wl"], p["bl"], p["Wr"], p["br"],
                       h_prev, act)
        num, den = _edge_stage(xl, xr, srcl, dstl, p["att"], hh, npg, epg)
        bias = p["bias"]
        h_prev, act = hh, True
    return _graph_ln_res(num, den, bias, x0, sp["ln_w"], sp["ln_b"], npg)


def _local_edges(ei, npg, epg_main, ngraphs):
    base = (jnp.arange(ngraphs, dtype=jnp.int32) * npg)[:, None]
    s = ei[0].astype(jnp.int32).reshape(ngraphs, epg_main) - base
    d = ei[1].astype(jnp.int32).reshape(ngraphs, epg_main) - base
    loop = jnp.broadcast_to(jnp.arange(npg, dtype=jnp.int32)[None, :],
                            (ngraphs, npg))
    srcl = jnp.concatenate([s, loop], axis=1).reshape(-1)
    dstl = jnp.concatenate([d, loop], axis=1).reshape(-1)
    return srcl, dstl


def kernel(params, circ_x, circ_edge_index, circ_batch, topo_x,
           topo_edge_index, topo_batch):
    p = params
    csrc, cdst = _local_edges(circ_edge_index, NC, ECG, B)
    tsrc, tdst = _local_edges(topo_edge_index, Q, ETG, B)

    x = p["circ_emb"][circ_x].reshape(-1, 2 * F)
    x = _stack(x, csrc, cdst, p["circ_stack"], 2 * F, H, NC, ECG + NC)
    circ_feat = x.reshape(-1, NC, 2 * F)[:, 0, :]

    cf = jnp.repeat(circ_feat, Q, axis=0)
    y = p["topo_emb"][topo_x].reshape(-1, F)
    y = jnp.concatenate([y, cf], axis=1)
    y = _stack(y, tsrc, tdst, p["topo_stack"], 3 * F, H, Q, ETG + Q)

    z = y.reshape(-1, Q, F)
    ii = jnp.repeat(jnp.arange(Q), Q)
    jj = jnp.tile(jnp.arange(Q), Q)
    pairs = jnp.stack([ii, jj], axis=1)
    zp = z[:, pairs].reshape(-1, 6 * F)
    out = _mlp(zp, p["mlp"])[:, 0]
    out = out.reshape(-1, Q, Q)
    out = (out + jnp.swapaxes(out, -1, -2)) / 2.0
    return out.reshape(-1, Q * Q)


# speedup vs baseline: 1.8853x; 1.8853x over previous
import functools
import jax
import jax.numpy as jnp
from jax import lax
from jax.experimental import pallas as pl
from jax.experimental.pallas import tpu as pltpu

B = 32
Q = 64
F = 64
H = 2
NC = 1024
NCN = B * NC
ECG = 16384
ETG = 512


def _lrelu(x, a):
    return jnp.maximum(x, a * x)


def _proj_body(num_ref, den_ref, bias_ref, wl_ref, bl_ref, wr_ref, br_ref,
               xl_ref, xr_ref, *, h_prev, act):
    x = num_ref[...]
    if act:
        den = den_ref[...]
        c = x.shape[1] // h_prev
        den_rep = jnp.concatenate(
            [jnp.broadcast_to(den[:, i:i + 1], (x.shape[0], c))
             for i in range(h_prev)], axis=1)
        x = _lrelu(x / den_rep + bias_ref[...], 0.01)
    xl_ref[...] = jnp.dot(x, wl_ref[...],
                          preferred_element_type=jnp.float32) + bl_ref[...]
    xr_ref[...] = jnp.dot(x, wr_ref[...],
                          preferred_element_type=jnp.float32) + br_ref[...]


def _proj(x, den, bias_prev, Wl, bl, Wr, br, h_prev, act):
    N, din = x.shape
    dout = Wl.shape[1]
    BLK = 512
    bias2 = bias_prev.reshape(1, din)
    bl2 = bl.reshape(1, dout)
    br2 = br.reshape(1, dout)
    grid = (N // BLK,)
    body = functools.partial(_proj_body, h_prev=h_prev, act=act)
    return pl.pallas_call(
        body,
        grid=grid,
        in_specs=[
            pl.BlockSpec((BLK, din), lambda i: (i, 0)),
            pl.BlockSpec((BLK, h_prev), lambda i: (i, 0)),
            pl.BlockSpec((1, din), lambda i: (0, 0)),
            pl.BlockSpec((din, dout), lambda i: (0, 0)),
            pl.BlockSpec((1, dout), lambda i: (0, 0)),
            pl.BlockSpec((din, dout), lambda i: (0, 0)),
            pl.BlockSpec((1, dout), lambda i: (0, 0)),
        ],
        out_specs=[
            pl.BlockSpec((BLK, dout), lambda i: (i, 0)),
            pl.BlockSpec((BLK, dout), lambda i: (i, 0)),
        ],
        out_shape=[
            jax.ShapeDtypeStruct((N, dout), jnp.float32),
            jax.ShapeDtypeStruct((N, dout), jnp.float32),
        ],
    )(x, den, bias2, Wl, bl2, Wr, br2)


def _ln_body(num_ref, den_ref, bias_ref, x_ref, w_ref, b_ref, o_ref):
    x = x_ref[...]
    g = _lrelu(num_ref[...] / den_ref[...] + bias_ref[...], 0.01)
    t = g + x
    mu = jnp.mean(t)
    var = jnp.mean((t - mu) * (t - mu))
    o_ref[...] = (t - mu) / jnp.sqrt(var + 1e-5) * w_ref[...] + b_ref[...] + x


def _graph_ln_res(num, den, bias, x, w, b, npg):
    N, fd = x.shape
    grid = (N // npg,)
    return pl.pallas_call(
        _ln_body,
        grid=grid,
        in_specs=[
            pl.BlockSpec((npg, fd), lambda i: (i, 0)),
            pl.BlockSpec((npg, 1), lambda i: (i, 0)),
            pl.BlockSpec((1, fd), lambda i: (0, 0)),
            pl.BlockSpec((npg, fd), lambda i: (i, 0)),
            pl.BlockSpec((1, fd), lambda i: (0, 0)),
            pl.BlockSpec((1, fd), lambda i: (0, 0)),
        ],
        out_specs=pl.BlockSpec((npg, fd), lambda i: (i, 0)),
        out_shape=jax.ShapeDtypeStruct((N, fd), jnp.float32),
    )(num, den, bias.reshape(1, fd), x, w.reshape(1, fd), b.reshape(1, fd))


def _mlp_body(zp_ref, w1_ref, b1_ref, w2_ref, b2_ref, lnw_ref, lnb_ref,
              w3_ref, b3_ref, w4_ref, b4_ref, o_ref):
    x = zp_ref[...]
    h1 = _lrelu(jnp.dot(x, w1_ref[...],
                        preferred_element_type=jnp.float32) + b1_ref[...], 0.01)
    h2 = _lrelu(jnp.dot(h1, w2_ref[...],
                        preferred_element_type=jnp.float32) + b2_ref[...], 0.01)
    mu = jnp.mean(h2, axis=1, keepdims=True)
    va = jnp.mean((h2 - mu) * (h2 - mu), axis=1, keepdims=True)
    h2 = (h2 - mu) / jnp.sqrt(va + 1e-5) * lnw_ref[...] + lnb_ref[...]
    h3 = _lrelu(jnp.dot(h2, w3_ref[...],
                        preferred_element_type=jnp.float32) + b3_ref[...], 0.01)
    o_ref[...] = jnp.dot(h3, w4_ref[...],
                         preferred_element_type=jnp.float32) + b4_ref[...]


def _mlp(zp, mp):
    R = zp.shape[0]
    BLK = 2048
    d1 = 6 * F
    d2 = 2 * F
    d3 = F // 2
    W4p = jnp.concatenate([mp["W4"], jnp.zeros((d3, 7), jnp.float32)], axis=1)
    b4p = jnp.concatenate([mp["b4"], jnp.zeros((7,), jnp.float32)])
    return pl.pallas_call(
        _mlp_body,
        grid=(R // BLK,),
        in_specs=[
            pl.BlockSpec((BLK, d1), lambda i: (i, 0)),
            pl.BlockSpec((d1, d1), lambda i: (0, 0)),
            pl.BlockSpec((1, d1), lambda i: (0, 0)),
            pl.BlockSpec((d1, d2), lambda i: (0, 0)),
            pl.BlockSpec((1, d2), lambda i: (0, 0)),
            pl.BlockSpec((1, d2), lambda i: (0, 0)),
            pl.BlockSpec((1, d2), lambda i: (0, 0)),
            pl.BlockSpec((d2, d3), lambda i: (0, 0)),
            pl.BlockSpec((1, d3), lambda i: (0, 0)),
            pl.BlockSpec((d3, 8), lambda i: (0, 0)),
            pl.BlockSpec((1, 8), lambda i: (0, 0)),
        ],
        out_specs=pl.BlockSpec((BLK, 8), lambda i: (i, 0)),
        out_shape=jax.ShapeDtypeStruct((R, 8), jnp.float32),
    )(zp, mp["W1"], mp["b1"].reshape(1, d1), mp["W2"], mp["b2"].reshape(1, d2),
      mp["lnw"].reshape(1, d2), mp["lnb"].reshape(1, d2), mp["W3"],
      mp["b3"].reshape(1, d3), W4p, b4p.reshape(1, 8))


def _sc_edge(xlT, xrT, srcl, dstl, attf, h, npg, epg, ngraphs):
    from jax.experimental.pallas import tpu_sc as plsc

    D, N = xlT.shape
    C = D // h
    NCH = D // 16
    MC = -(-NCH // 16)
    Eg = epg
    ES = -(-Eg // 16 // 16) * 16
    EgP = 16 * ES
    T = max(1, Eg // 1088)
    EB = Eg // T
    W = EgP + 256
    G = ngraphs // 2
    f32, i32 = jnp.float32, jnp.int32

    mesh = plsc.VectorSubcoreMesh(core_axis_name="c", subcore_axis_name="s")
    scratch = ([pltpu.VMEM((16, npg), f32)] * MC
               + [pltpu.VMEM((16, npg), f32)] * MC
               + [pltpu.VMEM((EB + 16,), i32),
                  pltpu.VMEM((EB + 16,), i32),
                  pltpu.VMEM((EB,), f32),
                  pltpu.VMEM((h, W), f32),
                  pltpu.VMEM((ES,), f32),
                  pltpu.VMEM((ES,), f32)]
               + [pltpu.VMEM((16, npg), f32)] * MC
               + [pltpu.VMEM((npg,), f32),
                  pltpu.SMEM((D,), f32),
                  pltpu.VMEM_SHARED((16, h, EgP), f32),
                  pltpu.VMEM_SHARED((h, W), f32),
                  pltpu.VMEM_SHARED((D,), f32)])

    def body(xl_hbm, xr_hbm, src_hbm, dst_hbm, att_hbm, num_hbm, den_hbm,
             *scr):
        xlb = scr[0:MC]
        xrb = scr[MC:2 * MC]
        srcb, dstb, plg, lgb, tmp, csum = scr[2 * MC:2 * MC + 6]
        accb = scr[2 * MC + 6:3 * MC + 6]
        denb, att_s, parts, wbuf, att_sp = scr[3 * MC + 6:]

        cc_ax = lax.axis_index("c")
        s_ax = lax.axis_index("s")
        iota = lax.iota(i32, 16)
        z16 = jnp.zeros((16,), f32)

        @pl.when(s_ax == 0)
        def _():
            pltpu.sync_copy(att_hbm, att_sp)
        plsc.subcore_barrier()
        pltpu.sync_copy(att_sp, att_s)

        def per_graph(gi, _):
            b = cc_ax * G + gi
            for k in range(MC):
                ch = s_ax + 16 * k

                @pl.when(ch < NCH)
                def _(k=k, ch=ch):
                    pltpu.sync_copy(
                        xl_hbm.at[pl.ds(ch * 16, 16), pl.ds(b * npg, npg)],
                        xlb[k])
                    pltpu.sync_copy(
                        xr_hbm.at[pl.ds(ch * 16, 16), pl.ds(b * npg, npg)],
                        xrb[k])

                def zacc(i, _, k=k):
                    for r in range(16):
                        accb[k][r, pl.ds(i * 16, 16)] = z16
                    return 0
                lax.fori_loop(0, npg // 16, zacc, 0)

            def zden(i, _):
                denb[pl.ds(i * 16, 16)] = z16
                return 0
            lax.fori_loop(0, npg // 16, zden, 0)

            def passA(t, _):
                e0 = b * Eg + t * EB
                pltpu.sync_copy(src_hbm.at[pl.ds(e0, EB)],
                                srcb.at[pl.ds(0, EB)])
                pltpu.sync_copy(dst_hbm.at[pl.ds(e0, EB)],
                                dstb.at[pl.ds(0, EB)])
                for k in range(MC):
                    ch = s_ax + 16 * k

                    @pl.when(ch < NCH)
                    def _(k=k, ch=ch):
                        def grp(i, _):
                            sv = srcb[pl.ds(i * 16, 16)]
                            dv = dstb[pl.ds(i * 16, 16)]
                            acc = z16
                            for ccol in range(16):
                                cidx = jnp.zeros((16,), i32) + ccol
                                gl = plsc.load_gather(xlb[k], [cidx, sv])
                                gr = plsc.load_gather(xrb[k], [cidx, dv])
                                t0 = gl + gr
                                t0 = jnp.maximum(t0, 0.2 * t0)
                                acc = acc + t0 * att_s[ch * 16 + ccol]
                            plg[pl.ds(i * 16, 16)] = acc
                            return 0
                        lax.fori_loop(0, EB // 16, grp, 0)
                        hd = (ch * 16) // C
                        pltpu.sync_copy(plg,
                                        parts.at[s_ax, hd,
                                                 pl.ds(t * EB, EB)])

                return 0
            lax.fori_loop(0, T, passA, 0)
            plsc.subcore_barrier()

            owners = [[j for j in range(16)
                       if any(j + 16 * k < NCH
                              and ((j + 16 * k) * 16) // C == hd
                              for k in range(MC))]
                      for hd in range(h)]
            es0 = s_ax * ES
            nv = jnp.clip((Eg - es0) // 16, 0, ES // 16)
            for hh in range(h):
                def zc(i, _):
                    csum[pl.ds(i * 16, 16)] = z16
                    return 0
                lax.fori_loop(0, ES // 16, zc, 0)
                for j in owners[hh]:
                    pltpu.sync_copy(parts.at[j, hh, pl.ds(es0, ES)], tmp)

                    def add16(i, _):
                        csum[pl.ds(i * 16, 16)] = (csum[pl.ds(i * 16, 16)]
                                                   + tmp[pl.ds(i * 16, 16)])
                        return 0
                    lax.fori_loop(0, ES // 16, add16, 0)
                pltpu.sync_copy(csum, wbuf.at[hh, pl.ds(es0, ES)])

                def mx(i, m):
                    return jnp.maximum(m, csum[pl.ds(i * 16, 16)])
                mv = lax.fori_loop(0, nv, mx, jnp.full((16,), -3e38, f32))
                plg[pl.ds(0, 16)] = jnp.zeros((16,), f32) + jnp.max(mv)
                pltpu.sync_copy(plg.at[pl.ds(0, 16)],
                                wbuf.at[hh, pl.ds(EgP + s_ax * 16, 16)])
            plsc.subcore_barrier()

            pltpu.sync_copy(wbuf, lgb)
            for hh in range(h):
                def mx2(i, m, hh=hh):
                    return jnp.maximum(m, lgb[hh, pl.ds(EgP + i * 16, 16)])
                mv = lax.fori_loop(0, 16, mx2, jnp.full((16,), -3e38, f32))
                mh = jnp.max(mv)

                def ex(i, _, hh=hh, mh=mh):
                    v = lgb[hh, pl.ds(i * 16, 16)]
                    lgb[hh, pl.ds(i * 16, 16)] = jnp.exp(v - mh)
                    return 0
                lax.fori_loop(0, Eg // 16, ex, 0)
            plsc.subcore_barrier()

            def passB(t, _):
                e0 = b * Eg + t * EB
                pltpu.sync_copy(src_hbm.at[pl.ds(e0, EB)],
                                srcb.at[pl.ds(0, EB)])
                pltpu.sync_copy(dst_hbm.at[pl.ds(e0, EB)],
                                dstb.at[pl.ds(0, EB)])

                def grp(i, _):
                    sv = srcb[pl.ds(i * 16, 16)]
                    dv = dstb[pl.ds(i * 16, 16)]
                    for k in range(MC):
                        ch = s_ax + 16 * k

                        @pl.when(ch < NCH)
                        def _(k=k, ch=ch):
                            hd = (ch * 16) // C
                            wv = lgb[hd, pl.ds(t * EB + i * 16, 16)]
                            for ccol in range(16):
                                cidx = jnp.zeros((16,), i32) + ccol
                                gl = plsc.load_gather(xlb[k], [cidx, sv])
                                plsc.addupdate_scatter(accb[k], [cidx, dv],
                                                       gl * wv)

                            @pl.when(ch * 16 % C == 0)
                            def _():
                                plsc.addupdate_scatter(denb, [dv], wv)
                    return 0
                lax.fori_loop(0, EB // 16, grp, 0)
                return 0
            lax.fori_loop(0, T, passB, 0)

            for k in range(MC):
                ch = s_ax + 16 * k

                @pl.when(ch < NCH)
                def _(k=k, ch=ch):
                    pltpu.sync_copy(
                        accb[k],
                        num_hbm.at[pl.ds(ch * 16, 16), pl.ds(b * npg, npg)])

                    @pl.when(ch * 16 % C == 0)
                    def _():
                        hd = (ch * 16) // C
                        pltpu.sync_copy(denb,
                                        den_hbm.at[hd, pl.ds(b * npg, npg)])
            plsc.subcore_barrier()
            return 0

        lax.fori_loop(0, G, per_graph, 0)

    run = pl.kernel(
        body,
        out_type=[jax.ShapeDtypeStruct((D, N), f32),
                  jax.ShapeDtypeStruct((h, N), f32)],
        mesh=mesh,
        scratch_types=scratch,
        compiler_params=pltpu.CompilerParams(use_tc_tiling_on_sc=False,
                                             needs_layout_passes=False),
    )
    num, den = run(xlT, xrT, srcl, dstl, attf)
    return num.T, den.T


def _edge_stage(xl, xr, srcl, dstl, att, h, npg, epg):
    ngraphs = xl.shape[0] // npg
    return _sc_edge(xl.T, xr.T, srcl, dstl, att.reshape(-1), h, npg, epg,
                    ngraphs)


def _stack(x0, srcl, dstl, sp, fd, h, npg, epg):
    num, den, bias = x0, jnp.ones((x0.shape[0], 1), jnp.float32), None
    h_prev, act = 1, False
    for li, p in enumerate(sp["convs"]):
        hh = h if li < 4 else 1
        bias_in = bias if bias is not None else jnp.zeros((num.shape[1],),
                                                          jnp.float32)
        xl, xr = _proj(num, den, bias_in, p["Wl"], p["bl"], p["Wr"], p["br"],
                       h_prev, act)
        num, den = _edge_stage(xl, xr, srcl, dstl, p["att"], hh, npg, epg)
        bias = p["bias"]
        h_prev, act = hh, True
    return _graph_ln_res(num, den, bias, x0, sp["ln_w"], sp["ln_b"], npg)


def _local_edges(ei, npg, epg_main, ngraphs):
    base = (jnp.arange(ngraphs, dtype=jnp.int32) * npg)[:, None]
    s = ei[0].astype(jnp.int32).reshape(ngraphs, epg_main) - base
    d = ei[1].astype(jnp.int32).reshape(ngraphs, epg_main) - base
    loop = jnp.broadcast_to(jnp.arange(npg, dtype=jnp.int32)[None, :],
                            (ngraphs, npg))
    srcl = jnp.concatenate([s, loop], axis=1).reshape(-1)
    dstl = jnp.concatenate([d, loop], axis=1).reshape(-1)
    return srcl, dstl


def kernel(params, circ_x, circ_edge_index, circ_batch, topo_x,
           topo_edge_index, topo_batch):
    p = params
    csrc, cdst = _local_edges(circ_edge_index, NC, ECG, B)
    tsrc, tdst = _local_edges(topo_edge_index, Q, ETG, B)

    x = p["circ_emb"][circ_x].reshape(-1, 2 * F)
    x = _stack(x, csrc, cdst, p["circ_stack"], 2 * F, H, NC, ECG + NC)
    circ_feat = x.reshape(-1, NC, 2 * F)[:, 0, :]

    cf = jnp.repeat(circ_feat, Q, axis=0)
    y = p["topo_emb"][topo_x].reshape(-1, F)
    y = jnp.concatenate([y, cf], axis=1)
    y = _stack(y, tsrc, tdst, p["topo_stack"], 3 * F, H, Q, ETG + Q)

    z = y.reshape(-1, Q, F)
    ii = jnp.repeat(jnp.arange(Q), Q)
    jj = jnp.tile(jnp.arange(Q), Q)
    pairs = jnp.stack([ii, jj], axis=1)
    zp = z[:, pairs].reshape(-1, 6 * F)
    out = _mlp(zp, p["mlp"])[:, 0]
    out = out.reshape(-1, Q, Q)
    out = (out + jnp.swapaxes(out, -1, -2)) / 2.0
    return out.reshape(-1, Q * Q)


# speedup vs baseline: 1.9013x; 1.0085x over previous
import functools
import jax
import jax.numpy as jnp
from jax import lax
from jax.experimental import pallas as pl
from jax.experimental.pallas import tpu as pltpu

B = 32
Q = 64
F = 64
H = 2
NC = 1024
NCN = B * NC
ECG = 16384
ETG = 512


def _lrelu(x, a):
    return jnp.maximum(x, a * x)


def _proj_body(num_ref, den_ref, bias_ref, wl_ref, bl_ref, wr_ref, br_ref,
               xl_ref, xr_ref, *, h_prev, act):
    x = num_ref[...]
    if act:
        den = den_ref[...]
        c = x.shape[1] // h_prev
        den_rep = jnp.concatenate(
            [jnp.broadcast_to(den[:, i:i + 1], (x.shape[0], c))
             for i in range(h_prev)], axis=1)
        x = _lrelu(x / den_rep + bias_ref[...], 0.01)
    xl_ref[...] = jnp.dot(x, wl_ref[...],
                          preferred_element_type=jnp.float32) + bl_ref[...]
    xr_ref[...] = jnp.dot(x, wr_ref[...],
                          preferred_element_type=jnp.float32) + br_ref[...]


def _proj(x, den, bias_prev, Wl, bl, Wr, br, h_prev, act):
    N, din = x.shape
    dout = Wl.shape[1]
    BLK = 512
    bias2 = bias_prev.reshape(1, din)
    bl2 = bl.reshape(1, dout)
    br2 = br.reshape(1, dout)
    grid = (N // BLK,)
    body = functools.partial(_proj_body, h_prev=h_prev, act=act)
    return pl.pallas_call(
        body,
        grid=grid,
        in_specs=[
            pl.BlockSpec((BLK, din), lambda i: (i, 0)),
            pl.BlockSpec((BLK, h_prev), lambda i: (i, 0)),
            pl.BlockSpec((1, din), lambda i: (0, 0)),
            pl.BlockSpec((din, dout), lambda i: (0, 0)),
            pl.BlockSpec((1, dout), lambda i: (0, 0)),
            pl.BlockSpec((din, dout), lambda i: (0, 0)),
            pl.BlockSpec((1, dout), lambda i: (0, 0)),
        ],
        out_specs=[
            pl.BlockSpec((BLK, dout), lambda i: (i, 0)),
            pl.BlockSpec((BLK, dout), lambda i: (i, 0)),
        ],
        out_shape=[
            jax.ShapeDtypeStruct((N, dout), jnp.float32),
            jax.ShapeDtypeStruct((N, dout), jnp.float32),
        ],
    )(x, den, bias2, Wl, bl2, Wr, br2)


def _ln_body(num_ref, den_ref, bias_ref, x_ref, w_ref, b_ref, o_ref):
    x = x_ref[...]
    g = _lrelu(num_ref[...] / den_ref[...] + bias_ref[...], 0.01)
    t = g + x
    mu = jnp.mean(t)
    var = jnp.mean((t - mu) * (t - mu))
    o_ref[...] = (t - mu) / jnp.sqrt(var + 1e-5) * w_ref[...] + b_ref[...] + x


def _graph_ln_res(num, den, bias, x, w, b, npg):
    N, fd = x.shape
    grid = (N // npg,)
    return pl.pallas_call(
        _ln_body,
        grid=grid,
        in_specs=[
            pl.BlockSpec((npg, fd), lambda i: (i, 0)),
            pl.BlockSpec((npg, 1), lambda i: (i, 0)),
            pl.BlockSpec((1, fd), lambda i: (0, 0)),
            pl.BlockSpec((npg, fd), lambda i: (i, 0)),
            pl.BlockSpec((1, fd), lambda i: (0, 0)),
            pl.BlockSpec((1, fd), lambda i: (0, 0)),
        ],
        out_specs=pl.BlockSpec((npg, fd), lambda i: (i, 0)),
        out_shape=jax.ShapeDtypeStruct((N, fd), jnp.float32),
    )(num, den, bias.reshape(1, fd), x, w.reshape(1, fd), b.reshape(1, fd))


def _mlp_body(zp_ref, w1_ref, b1_ref, w2_ref, b2_ref, lnw_ref, lnb_ref,
              w3_ref, b3_ref, w4_ref, b4_ref, o_ref):
    x = zp_ref[...]
    h1 = _lrelu(jnp.dot(x, w1_ref[...],
                        preferred_element_type=jnp.float32) + b1_ref[...], 0.01)
    h2 = _lrelu(jnp.dot(h1, w2_ref[...],
                        preferred_element_type=jnp.float32) + b2_ref[...], 0.01)
    mu = jnp.mean(h2, axis=1, keepdims=True)
    va = jnp.mean((h2 - mu) * (h2 - mu), axis=1, keepdims=True)
    h2 = (h2 - mu) / jnp.sqrt(va + 1e-5) * lnw_ref[...] + lnb_ref[...]
    h3 = _lrelu(jnp.dot(h2, w3_ref[...],
                        preferred_element_type=jnp.float32) + b3_ref[...], 0.01)
    o_ref[...] = jnp.dot(h3, w4_ref[...],
                         preferred_element_type=jnp.float32) + b4_ref[...]


def _mlp(zp, mp):
    R = zp.shape[0]
    BLK = 2048
    d1 = 6 * F
    d2 = 2 * F
    d3 = F // 2
    W4p = jnp.concatenate([mp["W4"], jnp.zeros((d3, 7), jnp.float32)], axis=1)
    b4p = jnp.concatenate([mp["b4"], jnp.zeros((7,), jnp.float32)])
    return pl.pallas_call(
        _mlp_body,
        grid=(R // BLK,),
        in_specs=[
            pl.BlockSpec((BLK, d1), lambda i: (i, 0)),
            pl.BlockSpec((d1, d1), lambda i: (0, 0)),
            pl.BlockSpec((1, d1), lambda i: (0, 0)),
            pl.BlockSpec((d1, d2), lambda i: (0, 0)),
            pl.BlockSpec((1, d2), lambda i: (0, 0)),
            pl.BlockSpec((1, d2), lambda i: (0, 0)),
            pl.BlockSpec((1, d2), lambda i: (0, 0)),
            pl.BlockSpec((d2, d3), lambda i: (0, 0)),
            pl.BlockSpec((1, d3), lambda i: (0, 0)),
            pl.BlockSpec((d3, 8), lambda i: (0, 0)),
            pl.BlockSpec((1, 8), lambda i: (0, 0)),
        ],
        out_specs=pl.BlockSpec((BLK, 8), lambda i: (i, 0)),
        out_shape=jax.ShapeDtypeStruct((R, 8), jnp.float32),
    )(zp, mp["W1"], mp["b1"].reshape(1, d1), mp["W2"], mp["b2"].reshape(1, d2),
      mp["lnw"].reshape(1, d2), mp["lnb"].reshape(1, d2), mp["W3"],
      mp["b3"].reshape(1, d3), W4p, b4p.reshape(1, 8))


def _sc_edge(xlT, xrT, srcl, dstl, attf, h, npg, epg, ngraphs):
    from jax.experimental.pallas import tpu_sc as plsc

    D, N = xlT.shape
    C = D // h
    NCH = D // 16
    MC = -(-NCH // 16)
    Eg = epg
    ES = -(-Eg // 16 // 16) * 16
    EgP = 16 * ES
    T = max(1, Eg // 1088)
    EB = Eg // T
    W = EgP + 256
    G = ngraphs // 2
    f32, i32 = jnp.float32, jnp.int32

    mesh = plsc.VectorSubcoreMesh(core_axis_name="c", subcore_axis_name="s")
    scratch = ([pltpu.VMEM((16, npg), f32)] * MC
               + [pltpu.VMEM((16, npg), f32)] * MC
               + [pltpu.VMEM((EB + 16,), i32),
                  pltpu.VMEM((EB + 16,), i32),
                  pltpu.VMEM((EB,), f32),
                  pltpu.VMEM((h, W), f32),
                  pltpu.VMEM((ES,), f32),
                  pltpu.VMEM((ES,), f32)]
               + [pltpu.VMEM((16, npg), f32)] * MC
               + [pltpu.VMEM((npg,), f32),
                  pltpu.SMEM((D,), f32),
                  pltpu.VMEM_SHARED((16, h, EgP), f32),
                  pltpu.VMEM_SHARED((h, W), f32),
                  pltpu.VMEM_SHARED((D,), f32)])

    def body(xl_hbm, xr_hbm, src_hbm, dst_hbm, att_hbm, num_hbm, den_hbm,
             *scr):
        xlb = scr[0:MC]
        xrb = scr[MC:2 * MC]
        srcb, dstb, plg, lgb, tmp, csum = scr[2 * MC:2 * MC + 6]
        accb = scr[2 * MC + 6:3 * MC + 6]
        denb, att_s, parts, wbuf, att_sp = scr[3 * MC + 6:]

        cc_ax = lax.axis_index("c")
        s_ax = lax.axis_index("s")
        iota = lax.iota(i32, 16)
        z16 = jnp.zeros((16,), f32)

        @pl.when(s_ax == 0)
        def _():
            pltpu.sync_copy(att_hbm, att_sp)
        plsc.subcore_barrier()
        pltpu.sync_copy(att_sp, att_s)

        def per_graph(gi, _):
            b = cc_ax * G + gi
            for k in range(MC):
                ch = s_ax + 16 * k

                @pl.when(ch < NCH)
                def _(k=k, ch=ch):
                    pltpu.sync_copy(
                        xl_hbm.at[pl.ds(ch * 16, 16), pl.ds(b * npg, npg)],
                        xlb[k])
                    pltpu.sync_copy(
                        xr_hbm.at[pl.ds(ch * 16, 16), pl.ds(b * npg, npg)],
                        xrb[k])

                def zacc(i, _, k=k):
                    for r in range(16):
                        accb[k][r, pl.ds(i * 16, 16)] = z16
                    return 0
                lax.fori_loop(0, npg // 16, zacc, 0)

            def zden(i, _):
                denb[pl.ds(i * 16, 16)] = z16
                return 0
            lax.fori_loop(0, npg // 16, zden, 0)

            def passA(t, _):
                e0 = b * Eg + t * EB
                pltpu.sync_copy(src_hbm.at[pl.ds(e0, EB)],
                                srcb.at[pl.ds(0, EB)])
                pltpu.sync_copy(dst_hbm.at[pl.ds(e0, EB)],
                                dstb.at[pl.ds(0, EB)])
                for k in range(MC):
                    ch = s_ax + 16 * k

                    @pl.when(ch < NCH)
                    def _(k=k, ch=ch):
                        def grp(i, _):
                            sv = srcb[pl.ds(i * 16, 16)]
                            dv = dstb[pl.ds(i * 16, 16)]
                            acc = z16
                            for ccol in range(16):
                                cidx = jnp.zeros((16,), i32) + ccol
                                gl = plsc.load_gather(xlb[k], [cidx, sv])
                                gr = plsc.load_gather(xrb[k], [cidx, dv])
                                t0 = gl + gr
                                t0 = jnp.maximum(t0, 0.2 * t0)
                                acc = acc + t0 * att_s[ch * 16 + ccol]
                            plg[pl.ds(i * 16, 16)] = acc
                            return 0
                        lax.fori_loop(0, EB // 16, grp, 0)
                        hd = (ch * 16) // C
                        pltpu.sync_copy(plg,
                                        parts.at[s_ax, hd,
                                                 pl.ds(t * EB, EB)])

                return 0
            lax.fori_loop(0, T, passA, 0)
            plsc.subcore_barrier()

            owners = [[j for j in range(16)
                       if any(j + 16 * k < NCH
                              and ((j + 16 * k) * 16) // C == hd
                              for k in range(MC))]
                      for hd in range(h)]
            es0 = s_ax * ES
            nv = jnp.clip((Eg - es0) // 16, 0, ES // 16)
            for hh in range(h):
                def zc(i, _):
                    csum[pl.ds(i * 16, 16)] = z16
                    return 0
                lax.fori_loop(0, ES // 16, zc, 0)
                for j in owners[hh]:
                    pltpu.sync_copy(parts.at[j, hh, pl.ds(es0, ES)], tmp)

                    def add16(i, _):
                        csum[pl.ds(i * 16, 16)] = (csum[pl.ds(i * 16, 16)]
                                                   + tmp[pl.ds(i * 16, 16)])
                        return 0
                    lax.fori_loop(0, ES // 16, add16, 0)
                pltpu.sync_copy(csum, wbuf.at[hh, pl.ds(es0, ES)])

                def mx(i, m):
                    return jnp.maximum(m, csum[pl.ds(i * 16, 16)])
                mv = lax.fori_loop(0, nv, mx, jnp.full((16,), -3e38, f32))
                plg[pl.ds(0, 16)] = jnp.zeros((16,), f32) + jnp.max(mv)
                pltpu.sync_copy(plg.at[pl.ds(0, 16)],
                                wbuf.at[hh, pl.ds(EgP + s_ax * 16, 16)])
            plsc.subcore_barrier()

            pltpu.sync_copy(wbuf, lgb)
            for hh in range(h):
                def mx2(i, m, hh=hh):
                    return jnp.maximum(m, lgb[hh, pl.ds(EgP + i * 16, 16)])
                mv = lax.fori_loop(0, 16, mx2, jnp.full((16,), -3e38, f32))
                mh = jnp.max(mv)

                def ex(i, _, hh=hh, mh=mh):
                    v = lgb[hh, pl.ds(i * 16, 16)]
                    lgb[hh, pl.ds(i * 16, 16)] = jnp.exp(v - mh)
                    return 0
                lax.fori_loop(0, Eg // 16, ex, 0)
            plsc.subcore_barrier()

            def passB(t, _):
                e0 = b * Eg + t * EB
                pltpu.sync_copy(src_hbm.at[pl.ds(e0, EB)],
                                srcb.at[pl.ds(0, EB)])
                pltpu.sync_copy(dst_hbm.at[pl.ds(e0, EB)],
                                dstb.at[pl.ds(0, EB)])

                def grp(i, _):
                    sv = srcb[pl.ds(i * 16, 16)]
                    dv = dstb[pl.ds(i * 16, 16)]
                    for k in range(MC):
                        ch = s_ax + 16 * k

                        @pl.when(ch < NCH)
                        def _(k=k, ch=ch):
                            hd = (ch * 16) // C
                            wv = lgb[hd, pl.ds(t * EB + i * 16, 16)]
                            for ccol in range(16):
                                cidx = jnp.zeros((16,), i32) + ccol
                                gl = plsc.load_gather(xlb[k], [cidx, sv])
                                plsc.addupdate_scatter(accb[k], [cidx, dv],
                                                       gl * wv)

                            @pl.when(ch * 16 % C == 0)
                            def _():
                                plsc.addupdate_scatter(denb, [dv], wv)
                    return 0
                lax.fori_loop(0, EB // 16, grp, 0)
                return 0
            lax.fori_loop(0, T, passB, 0)

            for k in range(MC):
                ch = s_ax + 16 * k

                @pl.when(ch < NCH)
                def _(k=k, ch=ch):
                    pltpu.sync_copy(
                        accb[k],
                        num_hbm.at[pl.ds(ch * 16, 16), pl.ds(b * npg, npg)])

                    @pl.when(ch * 16 % C == 0)
                    def _():
                        hd = (ch * 16) // C
                        pltpu.sync_copy(denb,
                                        den_hbm.at[hd, pl.ds(b * npg, npg)])
            plsc.subcore_barrier()
            return 0

        lax.fori_loop(0, G, per_graph, 0)

    run = pl.kernel(
        body,
        out_type=[jax.ShapeDtypeStruct((D, N), f32),
                  jax.ShapeDtypeStruct((h, N), f32)],
        mesh=mesh,
        scratch_types=scratch,
        compiler_params=pltpu.CompilerParams(use_tc_tiling_on_sc=False,
                                             needs_layout_passes=False),
    )
    num, den = run(xlT, xrT, srcl, dstl, attf)
    return num.T, den.T


def _sc_gather(table, idx):
    from jax.experimental.pallas import tpu_sc as plsc

    D = table.shape[1]
    Btot = idx.shape[0]
    bw = Btot // 32
    SB = min(bw, 1024)
    mesh = plsc.VectorSubcoreMesh(core_axis_name="c", subcore_axis_name="s")

    def body(table_hbm, idx_hbm, out_hbm, idx_v, rows_v, sem):
        wid = lax.axis_index("s") * 2 + lax.axis_index("c")
        base = wid * bw

        def bat(t, _):
            pltpu.sync_copy(idx_hbm.at[pl.ds(base + t * SB, SB)], idx_v)
            pltpu.async_copy(table_hbm.at[idx_v], rows_v, sem).wait()
            pltpu.sync_copy(rows_v, out_hbm.at[pl.ds(base + t * SB, SB)])
            return 0
        lax.fori_loop(0, bw // SB, bat, 0)

    run = pl.kernel(
        body,
        out_type=jax.ShapeDtypeStruct((Btot, D), jnp.float32),
        mesh=mesh,
        scratch_types=[pltpu.VMEM((SB,), jnp.int32),
                       pltpu.VMEM((SB, D), jnp.float32),
                       pltpu.SemaphoreType.DMA],
        compiler_params=pltpu.CompilerParams(use_tc_tiling_on_sc=False,
                                             needs_layout_passes=False),
    )
    return run(table, idx.astype(jnp.int32))


def _edge_stage(xl, xr, srcl, dstl, att, h, npg, epg):
    ngraphs = xl.shape[0] // npg
    return _sc_edge(xl.T, xr.T, srcl, dstl, att.reshape(-1), h, npg, epg,
                    ngraphs)


def _stack(x0, srcl, dstl, sp, fd, h, npg, epg):
    num, den, bias = x0, jnp.ones((x0.shape[0], 1), jnp.float32), None
    h_prev, act = 1, False
    for li, p in enumerate(sp["convs"]):
        hh = h if li < 4 else 1
        bias_in = bias if bias is not None else jnp.zeros((num.shape[1],),
                                                          jnp.float32)
        xl, xr = _proj(num, den, bias_in, p["Wl"], p["bl"], p["Wr"], p["br"],
                       h_prev, act)
        num, den = _edge_stage(xl, xr, srcl, dstl, p["att"], hh, npg, epg)
        bias = p["bias"]
        h_prev, act = hh, True
    return _graph_ln_res(num, den, bias, x0, sp["ln_w"], sp["ln_b"], npg)


def _local_edges(ei, npg, epg_main, ngraphs):
    base = (jnp.arange(ngraphs, dtype=jnp.int32) * npg)[:, None]
    s = ei[0].astype(jnp.int32).reshape(ngraphs, epg_main) - base
    d = ei[1].astype(jnp.int32).reshape(ngraphs, epg_main) - base
    loop = jnp.broadcast_to(jnp.arange(npg, dtype=jnp.int32)[None, :],
                            (ngraphs, npg))
    srcl = jnp.concatenate([s, loop], axis=1).reshape(-1)
    dstl = jnp.concatenate([d, loop], axis=1).reshape(-1)
    return srcl, dstl


def kernel(params, circ_x, circ_edge_index, circ_batch, topo_x,
           topo_edge_index, topo_batch):
    p = params
    csrc, cdst = _local_edges(circ_edge_index, NC, ECG, B)
    tsrc, tdst = _local_edges(topo_edge_index, Q, ETG, B)

    x = _sc_gather(p["circ_emb"], circ_x.reshape(-1)).reshape(-1, 2 * F)
    x = _stack(x, csrc, cdst, p["circ_stack"], 2 * F, H, NC, ECG + NC)
    circ_feat = x.reshape(-1, NC, 2 * F)[:, 0, :]

    cf = jnp.repeat(circ_feat, Q, axis=0)
    y = _sc_gather(p["topo_emb"], topo_x).reshape(-1, F)
    y = jnp.concatenate([y, cf], axis=1)
    y = _stack(y, tsrc, tdst, p["topo_stack"], 3 * F, H, Q, ETG + Q)

    z = y.reshape(-1, Q, F)
    ii = jnp.repeat(jnp.arange(Q), Q)
    jj = jnp.tile(jnp.arange(Q), Q)
    pairs = jnp.stack([ii, jj], axis=1)
    zp = z[:, pairs].reshape(-1, 6 * F)
    out = _mlp(zp, p["mlp"])[:, 0]
    out = out.reshape(-1, Q, Q)
    out = (out + jnp.swapaxes(out, -1, -2)) / 2.0
    return out.reshape(-1, Q * Q)
